# Initial kernel scaffold; baseline (speedup 1.0000x reference)
import jax
import jax.numpy as jnp
from jax.experimental import pallas as pl


def kernel(points):
    raise NotImplementedError("write your pallas kernel here")


# speedup vs baseline: 1.3354x; 1.3354x over previous
import functools

import jax
import jax.numpy as jnp
from jax import lax
from jax.experimental import pallas as pl
from jax.experimental.pallas import tpu as pltpu
from jax.experimental.pallas import tpu_sc as plsc

H, W = 300, 400
G = H * W
DUMP = G
GP = 120064
SEG = 7552
GS = 16 * SEG
NPTS = 1048576
NW = 32
PPW = NPTS // NW
CH = 1024
NCH = PPW // CH
NV = CH // 16
NR = CH // 128

X0, X1 = -20.0, 20.0
Y0, Y1 = -10.0, 30.0
Z0, Z1 = -3.0, 4.0
RES = 0.1
MAX_INTENSITY = 255.0
N_MAX = 128.0

_MESH = dict(core_axis_name="c", subcore_axis_name="s")
_PARAMS = pltpu.CompilerParams(needs_layout_passes=False)


def _coords(inbuf, v, lane, c1, c2, c3):
    rows = (v * 16 + lane) * 4
    x = plsc.load_gather(inbuf, [rows])
    y = plsc.load_gather(inbuf, [rows + c1])
    z = plsc.load_gather(inbuf, [rows + c2])
    w = plsc.load_gather(inbuf, [rows + c3])
    mask = (x >= X0) & (x < X1) & (y >= Y0) & (y < Y1)
    px = jnp.clip(((x - X0) / RES).astype(jnp.int32), 0, W - 1)
    py = jnp.clip(((y - Y0) / RES).astype(jnp.int32), 0, H - 1)
    flat = jnp.where(mask, py * W + px, DUMP)
    return flat, z, w


def _ids():
    cid = lax.axis_index("c")
    sid = lax.axis_index("s")
    wid = sid * 2 + cid
    return cid, sid, wid


def _consts():
    lane = lax.iota(jnp.int32, 16)
    c1 = jnp.full((16,), 1, jnp.int32)
    c2 = jnp.full((16,), 2, jnp.int32)
    c3 = jnp.full((16,), 3, jnp.int32)
    return lane, c1, c2, c3


def _sum_body(pts, sumout, inbuf, idxb, wvalb, onesb, zstage, sh_i, sh_c):
    cid, sid, wid = _ids()
    base = wid * PPW
    lane, c1, c2, c3 = _consts()
    zeros16 = jnp.zeros((16,), jnp.float32)
    ones16 = jnp.full((16,), 1.0, jnp.float32)

    def z_init(i, c):
        zstage[pl.ds(i * 16, 16)] = zeros16
        return c

    lax.fori_loop(0, SEG // 16, z_init, 0)
    pltpu.sync_copy(zstage, sh_i.at[pl.ds(sid * SEG, SEG)])
    pltpu.sync_copy(zstage, sh_c.at[pl.ds(sid * SEG, SEG)])
    plsc.subcore_barrier()

    for j in range(8):
        onesb[pl.ds(j * 16, 16)] = ones16

    def chunk_body(c, carry):
        pltpu.sync_copy(pts.at[pl.ds((base + c * CH) * 4, CH * 4)], inbuf)

        def vec_body(v, vcarry):
            flat, z, w = _coords(inbuf, v, lane, c1, c2, c3)
            r = v // 8
            off = (v % 8) * 16
            idxb[r, pl.ds(off, 16)] = flat
            wvalb[r, pl.ds(off, 16)] = w
            return vcarry

        lax.fori_loop(0, NV, vec_body, 0)

        for r in range(NR):
            pltpu.sync_copy(wvalb.at[r], sh_i.at[idxb.at[r]], add=True)
            pltpu.sync_copy(onesb, sh_c.at[idxb.at[r]], add=True)
        return carry

    lax.fori_loop(0, NCH, chunk_body, 0)

    plsc.subcore_barrier()
    pltpu.sync_copy(sh_i.at[pl.ds(sid * SEG, SEG)],
                    sumout.at[pl.ds((cid * 2 + 0) * GS + sid * SEG, SEG)])
    pltpu.sync_copy(sh_c.at[pl.ds(sid * SEG, SEG)],
                    sumout.at[pl.ds((cid * 2 + 1) * GS + sid * SEG, SEG)])


def _max_body(pts, maxout, inbuf, grid):
    cid, sid, wid = _ids()
    base = wid * PPW
    lane, c1, c2, c3 = _consts()
    ninf16 = jnp.full((16,), -jnp.inf, jnp.float32)

    def g_init(i, c):
        grid[pl.ds(i * 16, 16)] = ninf16
        return c

    lax.fori_loop(0, GP // 16, g_init, 0)

    def chunk_body(c, carry):
        pltpu.sync_copy(pts.at[pl.ds((base + c * CH) * 4, CH * 4)], inbuf)

        def vec_body(v, vcarry):
            flat, z, w = _coords(inbuf, v, lane, c1, c2, c3)

            cur0 = plsc.load_gather(grid, [flat])

            def m_cond(cur):
                return jnp.max(jnp.where(z > cur, 1, 0)) > 0

            def m_body(cur):
                plsc.store_scatter(grid, [flat], z, mask=z > cur)
                return plsc.load_gather(grid, [flat])

            lax.while_loop(m_cond, m_body, cur0)
            return vcarry

        lax.fori_loop(0, NV, vec_body, 0)
        return carry

    lax.fori_loop(0, NCH, chunk_body, 0)
    pltpu.sync_copy(grid, maxout.at[pl.ds(wid * GP, GP)])


def _sc_sums(points):
    run = functools.partial(
        pl.kernel,
        mesh=plsc.VectorSubcoreMesh(**_MESH),
        compiler_params=_PARAMS,
        out_type=jax.ShapeDtypeStruct((4 * GS,), jnp.float32),
        scratch_types=[
            pltpu.VMEM((CH * 4,), jnp.float32),
            pltpu.VMEM((NR, 128), jnp.int32),
            pltpu.VMEM((NR, 128), jnp.float32),
            pltpu.VMEM((128,), jnp.float32),
            pltpu.VMEM((SEG,), jnp.float32),
            pltpu.VMEM_SHARED((GS,), jnp.float32),
            pltpu.VMEM_SHARED((GS,), jnp.float32),
        ],
    )(_sum_body)
    return run(points)


def _sc_max(points):
    run = functools.partial(
        pl.kernel,
        mesh=plsc.VectorSubcoreMesh(**_MESH),
        compiler_params=_PARAMS,
        out_type=jax.ShapeDtypeStruct((NW * GP,), jnp.float32),
        scratch_types=[
            pltpu.VMEM((CH * 4,), jnp.float32),
            pltpu.VMEM((GP,), jnp.float32),
        ],
    )(_max_body)
    return run(points)


def _finalize_body(mref, sref, oref):
    h = jnp.max(mref[:, :G], axis=0)
    s = sref[...]
    isum = s[0, :G] + s[2, :G]
    cnt = s[1, :G] + s[3, :G]
    h = jnp.where(h == -jnp.inf, 0.0, h)
    denom = jnp.where(cnt > 0, cnt, 1.0)
    iavg = jnp.where(cnt > 0, isum / denom, 0.0)
    oref[0, :] = jnp.clip((h - Z0) / (Z1 - Z0), 0.0, 1.0)
    oref[1, :] = jnp.clip(iavg / MAX_INTENSITY, 0.0, 1.0)
    oref[2, :] = jnp.clip(jnp.log(1.0 + cnt) / jnp.log(1.0 + N_MAX), 0.0, 1.0)


def _finalize(maxg, sums):
    return pl.pallas_call(
        _finalize_body,
        out_shape=jax.ShapeDtypeStruct((3, G), jnp.float32),
    )(maxg, sums)


def kernel(points):
    flat_pts = points.reshape(-1)
    sums = _sc_sums(flat_pts)
    maxg = _sc_max(flat_pts)
    bev = _finalize(maxg.reshape(NW, GP), sums.reshape(4, GS))
    return bev.reshape(3, H, W)


# speedup vs baseline: 4.5237x; 3.3876x over previous
import functools

import jax
import jax.numpy as jnp
from jax import lax
from jax.experimental import pallas as pl
from jax.experimental.pallas import tpu as pltpu
from jax.experimental.pallas import tpu_sc as plsc

H, W = 300, 400
G = H * W
DUMP = G
GP = 120064
SEG = 7552
GS = 16 * SEG
NPTS = 1048576
NW = 32
PPW = NPTS // NW
CH = 512
NCH = PPW // CH
NV = CH // 16
NR = CH // 128

X0, X1 = -20.0, 20.0
Y0, Y1 = -10.0, 30.0
Z0, Z1 = -3.0, 4.0
RES = 0.1
MAX_INTENSITY = 255.0
N_MAX = 128.0

_MESH = dict(core_axis_name="c", subcore_axis_name="s")
_PARAMS = pltpu.CompilerParams(needs_layout_passes=False)


def _coords(xb, yb, v):
    x = xb[pl.ds(v * 16, 16)]
    y = yb[pl.ds(v * 16, 16)]
    mask = (x >= X0) & (x < X1) & (y >= Y0) & (y < Y1)
    px = jnp.clip(((x - X0) / RES).astype(jnp.int32), 0, W - 1)
    py = jnp.clip(((y - Y0) / RES).astype(jnp.int32), 0, H - 1)
    return jnp.where(mask, py * W + px, DUMP)


def _ids():
    cid = lax.axis_index("c")
    sid = lax.axis_index("s")
    wid = sid * 2 + cid
    return cid, sid, wid


def _sum_body(xs, ys, ws, sumout, xb, yb, idxb, wvalb, onesb, zstage, sh_i, sh_c):
    cid, sid, wid = _ids()
    base = wid * PPW
    zeros16 = jnp.zeros((16,), jnp.float32)
    ones16 = jnp.full((16,), 1.0, jnp.float32)

    def z_init(i, c):
        zstage[pl.ds(i * 16, 16)] = zeros16
        return c

    lax.fori_loop(0, SEG // 16, z_init, 0)
    pltpu.sync_copy(zstage, sh_i.at[pl.ds(sid * SEG, SEG)])
    pltpu.sync_copy(zstage, sh_c.at[pl.ds(sid * SEG, SEG)])
    plsc.subcore_barrier()

    for j in range(128 // 16):
        onesb[pl.ds(j * 16, 16)] = ones16

    def chunk_body(c, carry):
        b = base + c * CH
        pltpu.sync_copy(xs.at[pl.ds(b, CH)], xb)
        pltpu.sync_copy(ys.at[pl.ds(b, CH)], yb)
        pltpu.sync_copy(ws.at[pl.ds(b, CH)], wvalb)

        def vec_body(v, vcarry):
            flat = _coords(xb, yb, v)
            r = v // 8
            off = (v % 8) * 16
            idxb[r, pl.ds(off, 16)] = flat
            return vcarry

        lax.fori_loop(0, NV, vec_body, 0)

        for r in range(NR):
            pltpu.sync_copy(wvalb.at[pl.ds(r * 128, 128)],
                            sh_i.at[idxb.at[r]], add=True)
            pltpu.sync_copy(onesb, sh_c.at[idxb.at[r]], add=True)
        return carry

    lax.fori_loop(0, NCH, chunk_body, 0)

    plsc.subcore_barrier()
    pltpu.sync_copy(sh_i.at[pl.ds(sid * SEG, SEG)],
                    sumout.at[pl.ds((cid * 2 + 0) * GS + sid * SEG, SEG)])
    pltpu.sync_copy(sh_c.at[pl.ds(sid * SEG, SEG)],
                    sumout.at[pl.ds((cid * 2 + 1) * GS + sid * SEG, SEG)])


def _max_body(xs, ys, zs, maxout, xb, yb, zb, grid):
    cid, sid, wid = _ids()
    base = wid * PPW
    ninf16 = jnp.full((16,), -jnp.inf, jnp.float32)

    def g_init(i, c):
        grid[pl.ds(i * 16, 16)] = ninf16
        return c

    lax.fori_loop(0, GP // 16, g_init, 0)

    def chunk_body(c, carry):
        b = base + c * CH
        pltpu.sync_copy(xs.at[pl.ds(b, CH)], xb)
        pltpu.sync_copy(ys.at[pl.ds(b, CH)], yb)
        pltpu.sync_copy(zs.at[pl.ds(b, CH)], zb)

        def vec_body(v, vcarry):
            flat = _coords(xb, yb, v)
            z = zb[pl.ds(v * 16, 16)]

            cur0 = plsc.load_gather(grid, [flat])

            def m_cond(cur):
                return jnp.max(jnp.where(z > cur, 1, 0)) > 0

            def m_body(cur):
                plsc.store_scatter(grid, [flat], z, mask=z > cur)
                return plsc.load_gather(grid, [flat])

            lax.while_loop(m_cond, m_body, cur0)
            return vcarry

        lax.fori_loop(0, NV, vec_body, 0)
        return carry

    lax.fori_loop(0, NCH, chunk_body, 0)
    pltpu.sync_copy(grid, maxout.at[pl.ds(wid * GP, GP)])


def _sc_sums(xs, ys, ws):
    run = functools.partial(
        pl.kernel,
        mesh=plsc.VectorSubcoreMesh(**_MESH),
        compiler_params=_PARAMS,
        out_type=jax.ShapeDtypeStruct((4 * GS,), jnp.float32),
        scratch_types=[
            pltpu.VMEM((CH,), jnp.float32),
            pltpu.VMEM((CH,), jnp.float32),
            pltpu.VMEM((NR, 128), jnp.int32),
            pltpu.VMEM((CH,), jnp.float32),
            pltpu.VMEM((128,), jnp.float32),
            pltpu.VMEM((SEG,), jnp.float32),
            pltpu.VMEM_SHARED((GS,), jnp.float32),
            pltpu.VMEM_SHARED((GS,), jnp.float32),
        ],
    )(_sum_body)
    return run(xs, ys, ws)


def _sc_max(xs, ys, zs):
    run = functools.partial(
        pl.kernel,
        mesh=plsc.VectorSubcoreMesh(**_MESH),
        compiler_params=_PARAMS,
        out_type=jax.ShapeDtypeStruct((NW * GP,), jnp.float32),
        scratch_types=[
            pltpu.VMEM((CH,), jnp.float32),
            pltpu.VMEM((CH,), jnp.float32),
            pltpu.VMEM((CH,), jnp.float32),
            pltpu.VMEM((GP,), jnp.float32),
        ],
    )(_max_body)
    return run(xs, ys, zs)


def _finalize_body(mref, sref, oref):
    h = jnp.max(mref[:, :G], axis=0)
    s = sref[...]
    isum = s[0, :G] + s[2, :G]
    cnt = s[1, :G] + s[3, :G]
    h = jnp.where(h == -jnp.inf, 0.0, h)
    denom = jnp.where(cnt > 0, cnt, 1.0)
    iavg = jnp.where(cnt > 0, isum / denom, 0.0)
    oref[0, :] = jnp.clip((h - Z0) / (Z1 - Z0), 0.0, 1.0)
    oref[1, :] = jnp.clip(iavg / MAX_INTENSITY, 0.0, 1.0)
    oref[2, :] = jnp.clip(jnp.log(1.0 + cnt) / jnp.log(1.0 + N_MAX), 0.0, 1.0)


def _finalize(maxg, sums):
    return pl.pallas_call(
        _finalize_body,
        out_shape=jax.ShapeDtypeStruct((3, G), jnp.float32),
    )(maxg, sums)


def kernel(points):
    xs = points[:, 0]
    ys = points[:, 1]
    zs = points[:, 2]
    ws = points[:, 3]
    sums = _sc_sums(xs, ys, ws)
    maxg = _sc_max(xs, ys, zs)
    bev = _finalize(maxg.reshape(NW, GP), sums.reshape(4, GS))
    return bev.reshape(3, H, W)


# speedup vs baseline: 9.1416x; 2.0208x over previous
import functools

import jax
import jax.numpy as jnp
from jax import lax
from jax.experimental import pallas as pl
from jax.experimental.pallas import tpu as pltpu
from jax.experimental.pallas import tpu_sc as plsc

H, W = 300, 400
G = H * W
DUMP = G
GP = 120064
SEG = 7552
GS = 16 * SEG
NPTS = 1048576
NW = 32
PPW = NPTS // NW

CHS = 8192
NCHS = PPW // CHS
NRS = CHS // 128

CHM = 1024
NCHM = PPW // CHM

X0, X1 = -20.0, 20.0
Y0, Y1 = -10.0, 30.0
Z0, Z1 = -3.0, 4.0
RES = 0.1
MAX_INTENSITY = 255.0
N_MAX = 128.0

_MESH = dict(core_axis_name="c", subcore_axis_name="s")
_PARAMS = pltpu.CompilerParams(needs_layout_passes=False)


def _bin_index(x, y):
    mask = (x >= X0) & (x < X1) & (y >= Y0) & (y < Y1)
    px = jnp.clip(((x - X0) / RES).astype(jnp.int32), 0, W - 1)
    py = jnp.clip(((y - Y0) / RES).astype(jnp.int32), 0, H - 1)
    return jnp.where(mask, py * W + px, DUMP)


def _ids():
    cid = lax.axis_index("c")
    sid = lax.axis_index("s")
    wid = sid * 2 + cid
    return cid, sid, wid


def _sum_body(xs, ys, ws, sumout,
              xa, ya, wa, ia, xb, yb, wb, ib, onesb, zstage, sh_i, sh_c,
              sem_a, sem_b, ssem_a, ssem_b):
    cid, sid, wid = _ids()
    base = wid * PPW
    zeros16 = jnp.zeros((16,), jnp.float32)
    ones16 = jnp.full((16,), 1.0, jnp.float32)

    def z_init(i, c):
        for u in range(8):
            zstage[pl.ds(i * 128 + u * 16, 16)] = zeros16
        return c

    lax.fori_loop(0, SEG // 128, z_init, 0)
    pltpu.sync_copy(zstage, sh_i.at[pl.ds(sid * SEG, SEG)])
    pltpu.sync_copy(zstage, sh_c.at[pl.ds(sid * SEG, SEG)])
    plsc.subcore_barrier()

    for j in range(128 // 16):
        onesb[pl.ds(j * 16, 16)] = ones16

    def start_in(c, xd, yd, wd, sem):
        b = base + c * CHS
        pltpu.async_copy(xs.at[pl.ds(b, CHS)], xd, sem)
        pltpu.async_copy(ys.at[pl.ds(b, CHS)], yd, sem)
        pltpu.async_copy(ws.at[pl.ds(b, CHS)], wd, sem)

    def wait_in(xd, yd, wd, sem):
        pltpu.make_async_copy(xs.at[pl.ds(0, CHS)], xd, sem).wait()
        pltpu.make_async_copy(ys.at[pl.ds(0, CHS)], yd, sem).wait()
        pltpu.make_async_copy(ws.at[pl.ds(0, CHS)], wd, sem).wait()

    def compute(xd, yd, idxd):
        def vec_body(v, vc):
            x = xd[pl.ds(v * 16, 16)]
            y = yd[pl.ds(v * 16, 16)]
            flat = _bin_index(x, y)
            idxd[v // 8, pl.ds((v % 8) * 16, 16)] = flat
            return vc

        lax.fori_loop(0, CHS // 16, vec_body, 0)

    def fire(idxd, wd, ssem):
        def srow(r, c):
            pltpu.async_copy(wd.at[pl.ds(r * 128, 128)],
                             sh_i.at[idxd.at[r]], ssem, add=True)
            pltpu.async_copy(onesb, sh_c.at[idxd.at[r]], ssem, add=True)
            return c

        lax.fori_loop(0, NRS, srow, 0)

    def drain(idxd, wd, ssem):
        def drow(r, c):
            pltpu.make_async_copy(wd.at[pl.ds(r * 128, 128)],
                                  sh_i.at[idxd.at[r]], ssem).wait()
            pltpu.make_async_copy(onesb, sh_c.at[idxd.at[r]], ssem).wait()
            return c

        lax.fori_loop(0, NRS, drow, 0)

    start_in(0, xa, ya, wa, sem_a)

    def pair(q, c):
        c0 = 2 * q
        wait_in(xa, ya, wa, sem_a)
        compute(xa, ya, ia)

        @pl.when(q > 0)
        def _():
            drain(ib, wb, ssem_b)

        start_in(c0 + 1, xb, yb, wb, sem_b)
        fire(ia, wa, ssem_a)
        wait_in(xb, yb, wb, sem_b)
        compute(xb, yb, ib)
        drain(ia, wa, ssem_a)
        start_in(jnp.minimum(c0 + 2, NCHS - 1), xa, ya, wa, sem_a)
        fire(ib, wb, ssem_b)
        return c

    lax.fori_loop(0, NCHS // 2, pair, 0)
    wait_in(xa, ya, wa, sem_a)
    drain(ib, wb, ssem_b)

    plsc.subcore_barrier()
    pltpu.sync_copy(sh_i.at[pl.ds(sid * SEG, SEG)],
                    sumout.at[pl.ds((cid * 2 + 0) * GS + sid * SEG, SEG)])
    pltpu.sync_copy(sh_c.at[pl.ds(sid * SEG, SEG)],
                    sumout.at[pl.ds((cid * 2 + 1) * GS + sid * SEG, SEG)])


def _max_body(xs, ys, zs, maxout,
              xa, ya, za, xb, yb, zb, grid, sem_a, sem_b):
    cid, sid, wid = _ids()
    base = wid * PPW
    ninf16 = jnp.full((16,), -jnp.inf, jnp.float32)

    def g_init(i, c):
        for u in range(8):
            grid[pl.ds(i * 128 + u * 16, 16)] = ninf16
        return c

    lax.fori_loop(0, GP // 128, g_init, 0)

    def start_in(c, xd, yd, zd, sem):
        b = base + c * CHM
        pltpu.async_copy(xs.at[pl.ds(b, CHM)], xd, sem)
        pltpu.async_copy(ys.at[pl.ds(b, CHM)], yd, sem)
        pltpu.async_copy(zs.at[pl.ds(b, CHM)], zd, sem)

    def wait_in(xd, yd, zd, sem):
        pltpu.make_async_copy(xs.at[pl.ds(0, CHM)], xd, sem).wait()
        pltpu.make_async_copy(ys.at[pl.ds(0, CHM)], yd, sem).wait()
        pltpu.make_async_copy(zs.at[pl.ds(0, CHM)], zd, sem).wait()

    def compute(xd, yd, zd):
        def vec_body(v, vc):
            x = xd[pl.ds(v * 16, 16)]
            y = yd[pl.ds(v * 16, 16)]
            z = zd[pl.ds(v * 16, 16)]
            flat = _bin_index(x, y)

            cur0 = plsc.load_gather(grid, [flat])

            def m_cond(cur):
                return jnp.max(jnp.where(z > cur, 1, 0)) > 0

            def m_body(cur):
                plsc.store_scatter(grid, [flat], z, mask=z > cur)
                return plsc.load_gather(grid, [flat])

            lax.while_loop(m_cond, m_body, cur0)
            return vc

        lax.fori_loop(0, CHM // 16, vec_body, 0)

    start_in(0, xa, ya, za, sem_a)

    def pair(q, c):
        c0 = 2 * q
        start_in(c0 + 1, xb, yb, zb, sem_b)
        wait_in(xa, ya, za, sem_a)
        compute(xa, ya, za)
        start_in(jnp.minimum(c0 + 2, NCHM - 1), xa, ya, za, sem_a)
        wait_in(xb, yb, zb, sem_b)
        compute(xb, yb, zb)
        return c

    lax.fori_loop(0, NCHM // 2, pair, 0)
    wait_in(xa, ya, za, sem_a)
    pltpu.sync_copy(grid, maxout.at[pl.ds(wid * GP, GP)])


def _sc_sums(xs, ys, ws):
    run = functools.partial(
        pl.kernel,
        mesh=plsc.VectorSubcoreMesh(**_MESH),
        compiler_params=_PARAMS,
        out_type=jax.ShapeDtypeStruct((4 * GS,), jnp.float32),
        scratch_types=[
            pltpu.VMEM((CHS,), jnp.float32),
            pltpu.VMEM((CHS,), jnp.float32),
            pltpu.VMEM((CHS,), jnp.float32),
            pltpu.VMEM((NRS, 128), jnp.int32),
            pltpu.VMEM((CHS,), jnp.float32),
            pltpu.VMEM((CHS,), jnp.float32),
            pltpu.VMEM((CHS,), jnp.float32),
            pltpu.VMEM((NRS, 128), jnp.int32),
            pltpu.VMEM((128,), jnp.float32),
            pltpu.VMEM((SEG,), jnp.float32),
            pltpu.VMEM_SHARED((GS,), jnp.float32),
            pltpu.VMEM_SHARED((GS,), jnp.float32),
            pltpu.SemaphoreType.DMA,
            pltpu.SemaphoreType.DMA,
            pltpu.SemaphoreType.DMA,
            pltpu.SemaphoreType.DMA,
        ],
    )(_sum_body)
    return run(xs, ys, ws)


def _sc_max(xs, ys, zs):
    run = functools.partial(
        pl.kernel,
        mesh=plsc.VectorSubcoreMesh(**_MESH),
        compiler_params=_PARAMS,
        out_type=jax.ShapeDtypeStruct((NW * GP,), jnp.float32),
        scratch_types=[
            pltpu.VMEM((CHM,), jnp.float32),
            pltpu.VMEM((CHM,), jnp.float32),
            pltpu.VMEM((CHM,), jnp.float32),
            pltpu.VMEM((CHM,), jnp.float32),
            pltpu.VMEM((CHM,), jnp.float32),
            pltpu.VMEM((CHM,), jnp.float32),
            pltpu.VMEM((GP,), jnp.float32),
            pltpu.SemaphoreType.DMA,
            pltpu.SemaphoreType.DMA,
        ],
    )(_max_body)
    return run(xs, ys, zs)


def _finalize_body(mref, sref, oref):
    h = jnp.max(mref[:, :G], axis=0)
    s = sref[...]
    isum = s[0, :G] + s[2, :G]
    cnt = s[1, :G] + s[3, :G]
    h = jnp.where(h == -jnp.inf, 0.0, h)
    denom = jnp.where(cnt > 0, cnt, 1.0)
    iavg = jnp.where(cnt > 0, isum / denom, 0.0)
    oref[0, :] = jnp.clip((h - Z0) / (Z1 - Z0), 0.0, 1.0)
    oref[1, :] = jnp.clip(iavg / MAX_INTENSITY, 0.0, 1.0)
    oref[2, :] = jnp.clip(jnp.log(1.0 + cnt) / jnp.log(1.0 + N_MAX), 0.0, 1.0)


def _finalize(maxg, sums):
    return pl.pallas_call(
        _finalize_body,
        out_shape=jax.ShapeDtypeStruct((3, G), jnp.float32),
    )(maxg, sums)


def kernel(points):
    xs = points[:, 0]
    ys = points[:, 1]
    zs = points[:, 2]
    ws = points[:, 3]
    sums = _sc_sums(xs, ys, ws)
    maxg = _sc_max(xs, ys, zs)
    bev = _finalize(maxg.reshape(NW, GP), sums.reshape(4, GS))
    return bev.reshape(3, H, W)


# speedup vs baseline: 11.9825x; 1.3108x over previous
import functools

import jax
import jax.numpy as jnp
from jax import lax
from jax.experimental import pallas as pl
from jax.experimental.pallas import tpu as pltpu
from jax.experimental.pallas import tpu_sc as plsc

H, W = 300, 400
G = H * W
DUMP = G
GP = 120064
SEG = 7552
GS = 16 * SEG
NPTS = 1048576
NW = 32
PPW = NPTS // NW

CHS = 8192
NCHS = PPW // CHS
NRS = CHS // 128

CHM = 1024
NCHM = PPW // CHM

X0, X1 = -20.0, 20.0
Y0, Y1 = -10.0, 30.0
Z0, Z1 = -3.0, 4.0
RES = 0.1
MAX_INTENSITY = 255.0
N_MAX = 128.0

_MESH = dict(core_axis_name="c", subcore_axis_name="s")
_PARAMS = pltpu.CompilerParams(needs_layout_passes=False)


def _bin_index(x, y):
    mask = (x >= X0) & (x < X1) & (y >= Y0) & (y < Y1)
    px = jnp.clip(((x - X0) / RES).astype(jnp.int32), 0, W - 1)
    py = jnp.clip(((y - Y0) / RES).astype(jnp.int32), 0, H - 1)
    return jnp.where(mask, py * W + px, DUMP)


def _ids():
    cid = lax.axis_index("c")
    sid = lax.axis_index("s")
    wid = sid * 2 + cid
    return cid, sid, wid


def _sum_body(xs, ys, ws, sumout,
              xa, ya, wa, ia, xb, yb, wb, ib, onesb, zstage, sh_i, sh_c,
              sem_a, sem_b, ssem_a, ssem_b):
    cid, sid, wid = _ids()
    base = wid * PPW
    zeros16 = jnp.zeros((16,), jnp.float32)
    ones16 = jnp.full((16,), 1.0, jnp.float32)

    def z_init(i, c):
        for u in range(8):
            zstage[pl.ds(i * 128 + u * 16, 16)] = zeros16
        return c

    lax.fori_loop(0, SEG // 128, z_init, 0)
    pltpu.sync_copy(zstage, sh_i.at[pl.ds(sid * SEG, SEG)])
    pltpu.sync_copy(zstage, sh_c.at[pl.ds(sid * SEG, SEG)])
    plsc.subcore_barrier()

    for j in range(128 // 16):
        onesb[pl.ds(j * 16, 16)] = ones16

    def start_in(c, xd, yd, wd, sem):
        b = base + c * CHS
        pltpu.async_copy(xs.at[pl.ds(b, CHS)], xd, sem)
        pltpu.async_copy(ys.at[pl.ds(b, CHS)], yd, sem)
        pltpu.async_copy(ws.at[pl.ds(b, CHS)], wd, sem)

    def wait_in(xd, yd, wd, sem):
        pltpu.make_async_copy(xs.at[pl.ds(0, CHS)], xd, sem).wait()
        pltpu.make_async_copy(ys.at[pl.ds(0, CHS)], yd, sem).wait()
        pltpu.make_async_copy(ws.at[pl.ds(0, CHS)], wd, sem).wait()

    def compute(xd, yd, idxd):
        def vec_body(v, vc):
            x = xd[pl.ds(v * 16, 16)]
            y = yd[pl.ds(v * 16, 16)]
            flat = _bin_index(x, y)
            idxd[v // 8, pl.ds((v % 8) * 16, 16)] = flat
            return vc

        lax.fori_loop(0, CHS // 16, vec_body, 0)

    def fire(idxd, wd, ssem):
        def srow(r, c):
            pltpu.async_copy(wd.at[pl.ds(r * 128, 128)],
                             sh_i.at[idxd.at[r]], ssem, add=True)
            pltpu.async_copy(onesb, sh_c.at[idxd.at[r]], ssem, add=True)
            return c

        lax.fori_loop(0, NRS, srow, 0)

    def drain(idxd, wd, ssem):
        def drow(r, c):
            pltpu.make_async_copy(wd.at[pl.ds(r * 128, 128)],
                                  sh_i.at[idxd.at[r]], ssem).wait()
            pltpu.make_async_copy(onesb, sh_c.at[idxd.at[r]], ssem).wait()
            return c

        lax.fori_loop(0, NRS, drow, 0)

    start_in(0, xa, ya, wa, sem_a)

    def pair(q, c):
        c0 = 2 * q
        wait_in(xa, ya, wa, sem_a)
        compute(xa, ya, ia)

        @pl.when(q > 0)
        def _():
            drain(ib, wb, ssem_b)

        start_in(c0 + 1, xb, yb, wb, sem_b)
        fire(ia, wa, ssem_a)
        wait_in(xb, yb, wb, sem_b)
        compute(xb, yb, ib)
        drain(ia, wa, ssem_a)
        start_in(jnp.minimum(c0 + 2, NCHS - 1), xa, ya, wa, sem_a)
        fire(ib, wb, ssem_b)
        return c

    lax.fori_loop(0, NCHS // 2, pair, 0)
    wait_in(xa, ya, wa, sem_a)
    drain(ib, wb, ssem_b)

    plsc.subcore_barrier()
    pltpu.sync_copy(sh_i.at[pl.ds(sid * SEG, SEG)],
                    sumout.at[pl.ds((cid * 2 + 0) * GS + sid * SEG, SEG)])
    pltpu.sync_copy(sh_c.at[pl.ds(sid * SEG, SEG)],
                    sumout.at[pl.ds((cid * 2 + 1) * GS + sid * SEG, SEG)])


def _max_body(xs, ys, zs, maxout,
              xa, ya, za, xb, yb, zb, grid, sem_a, sem_b):
    cid, sid, wid = _ids()
    base = wid * PPW
    ninf16 = jnp.full((16,), -jnp.inf, jnp.float32)

    def g_init(i, c):
        for u in range(8):
            grid[pl.ds(i * 128 + u * 16, 16)] = ninf16
        return c

    lax.fori_loop(0, GP // 128, g_init, 0)

    lane = lax.iota(jnp.int32, 16)
    shift_idx = [jnp.maximum(lane - d, 0) for d in (1, 2, 4, 8)]
    next_idx = jnp.minimum(lane + 1, 15)
    is_hi = lane == 15

    def start_in(c, xd, yd, zd, sem):
        b = base + c * CHM
        pltpu.async_copy(xs.at[pl.ds(b, CHM)], xd, sem)
        pltpu.async_copy(ys.at[pl.ds(b, CHM)], yd, sem)
        pltpu.async_copy(zs.at[pl.ds(b, CHM)], zd, sem)

    def wait_in(xd, yd, zd, sem):
        pltpu.make_async_copy(xs.at[pl.ds(0, CHM)], xd, sem).wait()
        pltpu.make_async_copy(ys.at[pl.ds(0, CHM)], yd, sem).wait()
        pltpu.make_async_copy(zs.at[pl.ds(0, CHM)], zd, sem).wait()

    def compute(xd, yd, zd):
        def vec_body(v, vc):
            x = xd[pl.ds(v * 16, 16)]
            y = yd[pl.ds(v * 16, 16)]
            z = zd[pl.ds(v * 16, 16)]
            flat = _bin_index(x, y)

            k, zv = plsc.sort_key_val(flat, z)
            for idxd in shift_idx:
                kshift = jnp.take_along_axis(k, idxd, axis=0)
                zshift = jnp.take_along_axis(zv, idxd, axis=0)
                zv = jnp.where(kshift == k, jnp.maximum(zv, zshift), zv)
            kn = jnp.take_along_axis(k, next_idx, axis=0)
            last = (kn != k) | is_hi
            cur = plsc.load_gather(grid, [k])
            plsc.store_scatter(grid, [k], jnp.maximum(cur, zv), mask=last)
            return vc

        lax.fori_loop(0, CHM // 16, vec_body, 0)

    start_in(0, xa, ya, za, sem_a)

    def pair(q, c):
        c0 = 2 * q
        start_in(c0 + 1, xb, yb, zb, sem_b)
        wait_in(xa, ya, za, sem_a)
        compute(xa, ya, za)
        start_in(jnp.minimum(c0 + 2, NCHM - 1), xa, ya, za, sem_a)
        wait_in(xb, yb, zb, sem_b)
        compute(xb, yb, zb)
        return c

    lax.fori_loop(0, NCHM // 2, pair, 0)
    wait_in(xa, ya, za, sem_a)
    pltpu.sync_copy(grid, maxout.at[pl.ds(wid * GP, GP)])


def _sc_sums(xs, ys, ws):
    run = functools.partial(
        pl.kernel,
        mesh=plsc.VectorSubcoreMesh(**_MESH),
        compiler_params=_PARAMS,
        out_type=jax.ShapeDtypeStruct((4 * GS,), jnp.float32),
        scratch_types=[
            pltpu.VMEM((CHS,), jnp.float32),
            pltpu.VMEM((CHS,), jnp.float32),
            pltpu.VMEM((CHS,), jnp.float32),
            pltpu.VMEM((NRS, 128), jnp.int32),
            pltpu.VMEM((CHS,), jnp.float32),
            pltpu.VMEM((CHS,), jnp.float32),
            pltpu.VMEM((CHS,), jnp.float32),
            pltpu.VMEM((NRS, 128), jnp.int32),
            pltpu.VMEM((128,), jnp.float32),
            pltpu.VMEM((SEG,), jnp.float32),
            pltpu.VMEM_SHARED((GS,), jnp.float32),
            pltpu.VMEM_SHARED((GS,), jnp.float32),
            pltpu.SemaphoreType.DMA,
            pltpu.SemaphoreType.DMA,
            pltpu.SemaphoreType.DMA,
            pltpu.SemaphoreType.DMA,
        ],
    )(_sum_body)
    return run(xs, ys, ws)


def _sc_max(xs, ys, zs):
    run = functools.partial(
        pl.kernel,
        mesh=plsc.VectorSubcoreMesh(**_MESH),
        compiler_params=_PARAMS,
        out_type=jax.ShapeDtypeStruct((NW * GP,), jnp.float32),
        scratch_types=[
            pltpu.VMEM((CHM,), jnp.float32),
            pltpu.VMEM((CHM,), jnp.float32),
            pltpu.VMEM((CHM,), jnp.float32),
            pltpu.VMEM((CHM,), jnp.float32),
            pltpu.VMEM((CHM,), jnp.float32),
            pltpu.VMEM((CHM,), jnp.float32),
            pltpu.VMEM((GP,), jnp.float32),
            pltpu.SemaphoreType.DMA,
            pltpu.SemaphoreType.DMA,
        ],
    )(_max_body)
    return run(xs, ys, zs)


def _finalize_body(mref, sref, oref):
    h = jnp.max(mref[:, :G], axis=0)
    s = sref[...]
    isum = s[0, :G] + s[2, :G]
    cnt = s[1, :G] + s[3, :G]
    h = jnp.where(h == -jnp.inf, 0.0, h)
    denom = jnp.where(cnt > 0, cnt, 1.0)
    iavg = jnp.where(cnt > 0, isum / denom, 0.0)
    oref[0, :] = jnp.clip((h - Z0) / (Z1 - Z0), 0.0, 1.0)
    oref[1, :] = jnp.clip(iavg / MAX_INTENSITY, 0.0, 1.0)
    oref[2, :] = jnp.clip(jnp.log(1.0 + cnt) / jnp.log(1.0 + N_MAX), 0.0, 1.0)


def _finalize(maxg, sums):
    return pl.pallas_call(
        _finalize_body,
        out_shape=jax.ShapeDtypeStruct((3, G), jnp.float32),
    )(maxg, sums)


def kernel(points):
    xs = points[:, 0]
    ys = points[:, 1]
    zs = points[:, 2]
    ws = points[:, 3]
    sums = _sc_sums(xs, ys, ws)
    maxg = _sc_max(xs, ys, zs)
    bev = _finalize(maxg.reshape(NW, GP), sums.reshape(4, GS))
    return bev.reshape(3, H, W)
